# baseline (speedup 1.0000x reference)
import functools

import jax
import jax.numpy as jnp
from jax import lax
from jax.experimental import pallas as pl
from jax.experimental.pallas import tpu as pltpu
from jax.experimental.pallas import tpu_sc as plsc

_LANES = 16
_NW = 16


def _hat_mask_body(emb_hbm, aux_hbm, out_hbm, aux_v, row_v, sem_a, sem_g):
    slc = row_v.shape[1]
    wid = lax.axis_index("s")
    base = wid * slc
    pltpu.async_copy(aux_hbm, aux_v, sem_a).wait()
    idx_ref = aux_v.at[pl.ds(_LANES, 1)]
    cp_g = pltpu.async_copy(emb_hbm.at[idx_ref, pl.ds(base, slc)], row_v,
                            sem_g)
    sv = plsc.bitcast(aux_v[pl.ds(0, _LANES)], jnp.float32)
    cp_g.wait()
    for j in range(slc // _LANES):
        x = row_v[0, pl.ds(j * _LANES, _LANES)]
        row_v[0, pl.ds(j * _LANES, _LANES)] = 1.0 / (1.0 + jnp.exp(-(sv * x)))
    pltpu.sync_copy(row_v.at[0], out_hbm.at[pl.ds(base, slc)])


def kernel(embeddings, task_id, s):
    n_tasks, n_units = embeddings.shape
    slc = n_units // _NW
    s_bits = jax.lax.bitcast_convert_type(jnp.float32(s), jnp.int32)
    aux = jnp.concatenate([
        jnp.full((_LANES,), s_bits, dtype=jnp.int32),
        jnp.full((_LANES,), task_id, dtype=jnp.int32),
    ])

    f = functools.partial(
        pl.kernel,
        out_type=jax.ShapeDtypeStruct((n_units,), jnp.float32),
        mesh=plsc.VectorSubcoreMesh(core_axis_name="c", subcore_axis_name="s",
                                    num_cores=1),
        compiler_params=pltpu.CompilerParams(use_tc_tiling_on_sc=True,
                                             needs_layout_passes=False),
        scratch_types=[
            pltpu.VMEM((2 * _LANES,), jnp.int32),
            pltpu.VMEM((1, slc), jnp.float32),
            pltpu.SemaphoreType.DMA,
            pltpu.SemaphoreType.DMA,
        ],
    )(_hat_mask_body)
    return f(embeddings, aux)


# speedup vs baseline: 1.0080x; 1.0080x over previous
import functools

import jax
import jax.numpy as jnp
from jax import lax
from jax.experimental import pallas as pl
from jax.experimental.pallas import tpu as pltpu
from jax.experimental.pallas import tpu_sc as plsc

_LANES = 16
_NW = 16


def _hat_mask_body(emb_hbm, idx_hbm, s_hbm, out_hbm, idx_v, s_v, row_v,
                   out_v, sem_i, sem_s, sem_g):
    slc = out_v.shape[0]
    wid = lax.axis_index("s")
    cp_i = pltpu.async_copy(idx_hbm, idx_v, sem_i)
    cp_s = pltpu.async_copy(s_hbm, s_v, sem_s)
    cp_i.wait()
    base = wid * slc
    cp_g = pltpu.async_copy(emb_hbm.at[idx_v, pl.ds(base, slc)], row_v, sem_g)
    cp_s.wait()
    cp_g.wait()
    sv = s_v[...]
    for j in range(slc // _LANES):
        x = row_v[0, pl.ds(j * _LANES, _LANES)]
        out_v[pl.ds(j * _LANES, _LANES)] = 1.0 / (1.0 + jnp.exp(-(sv * x)))
    pltpu.sync_copy(out_v, out_hbm.at[pl.ds(base, slc)])


def kernel(embeddings, task_id, s):
    n_tasks, n_units = embeddings.shape
    slc = n_units // _NW
    idx = jnp.reshape(jnp.int32(task_id), (1,))
    s_vec = jnp.full((_LANES,), s, dtype=jnp.float32)

    f = functools.partial(
        pl.kernel,
        out_type=jax.ShapeDtypeStruct((n_units,), jnp.float32),
        mesh=plsc.VectorSubcoreMesh(core_axis_name="c", subcore_axis_name="s",
                                    num_cores=1),
        compiler_params=pltpu.CompilerParams(use_tc_tiling_on_sc=True),
        scratch_types=[
            pltpu.VMEM((1,), jnp.int32),
            pltpu.VMEM((_LANES,), jnp.float32),
            pltpu.VMEM((1, slc), jnp.float32),
            pltpu.VMEM((slc,), jnp.float32),
            pltpu.SemaphoreType.DMA,
            pltpu.SemaphoreType.DMA,
            pltpu.SemaphoreType.DMA,
        ],
    )(_hat_mask_body)
    return f(embeddings, idx, s_vec)
